# baseline (speedup 1.0000x reference)
import jax
import jax.numpy as jnp
from jax import lax
from jax.experimental import pallas as pl
from jax.experimental.pallas import tpu as pltpu
from jax.experimental.pallas import tpu_sc as plsc

POS_TH = 0.5
NEG_TH = 0.4
EPS = 1e-7
LOG_EPS = -16.11809565095832
BETA = 1.0 / 9.0

_INTERPRET = False

B, A = 8, 16384
T1 = 4096
T2 = 16384
NGT = 50
NC = 81

NSC, NSUB = 2, 16
NW = NSC * NSUB
WPB = NW // B
APW = A // WPB
LANE = 16
NGT_PAD = 64


def _sc_match(anch_hbm, gt_hbm, conf_hbm, mbox_hbm,
              ax1_v, ay1_v, ax2_v, ay2_v, gt_v,
              conf_v, m0_v, m1_v, m2_v, m3_v):
    cid = lax.axis_index("c")
    sid = lax.axis_index("s")
    wid = sid * NSC + cid
    b = wid // WPB
    base = (wid % WPB) * APW

    pltpu.sync_copy(anch_hbm.at[0, pl.ds(base, APW)], ax1_v)
    pltpu.sync_copy(anch_hbm.at[1, pl.ds(base, APW)], ay1_v)
    pltpu.sync_copy(anch_hbm.at[2, pl.ds(base, APW)], ax2_v)
    pltpu.sync_copy(anch_hbm.at[3, pl.ds(base, APW)], ay2_v)
    pltpu.sync_copy(gt_hbm.at[b], gt_v)

    gsc = []
    for r in range(6):
        chunks = [gt_v[pl.ds(r * NGT_PAD + c * LANE, LANE)]
                  for c in range(NGT_PAD // LANE)]
        gsc.append([chunks[g // LANE][g % LANE] for g in range(NGT)])

    def chunk(i, carry):
        for u in range(2):
            s = i * (2 * LANE) + u * LANE
            ax1 = ax1_v[pl.ds(s, LANE)]
            ay1 = ay1_v[pl.ds(s, LANE)]
            ax2 = ax2_v[pl.ds(s, LANE)]
            ay2 = ay2_v[pl.ds(s, LANE)]
            area_a = (ax2 - ax1) * (ay2 - ay1)

            z = jnp.zeros((LANE,), jnp.float32)
            best = jnp.full((LANE,), -1.0, jnp.float32)
            labv = z
            mx1 = z
            my1 = z
            mx2 = z
            my2 = z
            for g in range(NGT):
                gx1 = gsc[0][g]
                gy1 = gsc[1][g]
                gx2 = gsc[2][g]
                gy2 = gsc[3][g]
                garea = gsc[4][g]
                glab = gsc[5][g]
                wx = jnp.maximum(jnp.minimum(ax2, gx2) - jnp.maximum(ax1, gx1),
                                 0.0)
                wy = jnp.maximum(jnp.minimum(ay2, gy2) - jnp.maximum(ay1, gy1),
                                 0.0)
                inter = wx * wy
                iou = inter / (area_a + (garea - inter))
                upd = iou > best
                best = jnp.where(upd, iou, best)
                labv = jnp.where(upd, glab, labv)
                mx1 = jnp.where(upd, gx1, mx1)
                my1 = jnp.where(upd, gy1, my1)
                mx2 = jnp.where(upd, gx2, mx2)
                my2 = jnp.where(upd, gy2, my2)

            conf = jnp.where(best < POS_TH, -1.0, labv)
            conf = jnp.where(best < NEG_TH, 0.0, conf)
            conf_v[pl.ds(s, LANE)] = conf
            m0_v[pl.ds(s, LANE)] = mx1
            m1_v[pl.ds(s, LANE)] = my1
            m2_v[pl.ds(s, LANE)] = mx2
            m3_v[pl.ds(s, LANE)] = my2
        return carry

    lax.fori_loop(0, APW // (2 * LANE), chunk, 0)

    pltpu.sync_copy(conf_v, conf_hbm.at[b, 0, pl.ds(base, APW)])
    pltpu.sync_copy(m0_v, mbox_hbm.at[b, 0, pl.ds(base, APW)])
    pltpu.sync_copy(m1_v, mbox_hbm.at[b, 1, pl.ds(base, APW)])
    pltpu.sync_copy(m2_v, mbox_hbm.at[b, 2, pl.ds(base, APW)])
    pltpu.sync_copy(m3_v, mbox_hbm.at[b, 3, pl.ds(base, APW)])


def _loc_body(a_ref, ploc_ref, conf_ref, mbox_ref, scal_ref, acc_ref):
    b = pl.program_id(0)
    t = pl.program_id(1)
    first = jnp.logical_and(b == 0, t == 0)
    last = jnp.logical_and(b == pl.num_programs(0) - 1,
                           t == pl.num_programs(1) - 1)

    @pl.when(first)
    def _():
        acc_ref[0] = 0.0
        acc_ref[1] = 0.0

    conf = conf_ref[0, 0:1, :]
    pos = (conf > 0.0).astype(jnp.float32)
    mx1 = mbox_ref[0, 0:1, :]
    my1 = mbox_ref[0, 1:2, :]
    mx2 = mbox_ref[0, 2:3, :]
    my2 = mbox_ref[0, 3:4, :]
    acx = a_ref[4:5, :]
    acy = a_ref[5:6, :]
    aw = a_ref[6:7, :]
    ah = a_ref[7:8, :]
    l0 = ((mx1 + mx2) * 0.5 - acx) / (aw * 0.1)
    l1 = ((my1 + my2) * 0.5 - acy) / (ah * 0.1)
    l2 = jnp.log(jnp.maximum(mx2 - mx1, 1e-6) / aw) * 5.0
    l3 = jnp.log(jnp.maximum(my2 - my1, 1e-6) / ah) * 5.0

    sl1 = jnp.zeros((1, T1), jnp.float32)
    for i, l in enumerate((l0, l1, l2, l3)):
        n = jnp.abs(ploc_ref[0, i:i + 1, :] - l)
        sl1 = sl1 + jnp.where(n < BETA, n * n * (0.5 / BETA), n - 0.5 * BETA)
    acc_ref[0] += jnp.sum(sl1 * pos)
    acc_ref[1] += jnp.sum(pos)

    @pl.when(last)
    def _():
        scal_ref[0:1, 0:1] = jnp.full((1, 1), acc_ref[0])
        scal_ref[0:1, 1:2] = jnp.full((1, 1), acc_ref[1])


def _loss_body(conf_cls_ref, lab_ref, scal_ref, out1_ref, out2_ref, acc_ref):
    b = pl.program_id(0)
    t = pl.program_id(1)
    first = jnp.logical_and(b == 0, t == 0)
    last = jnp.logical_and(b == pl.num_programs(0) - 1,
                           t == pl.num_programs(1) - 1)

    @pl.when(first)
    def _():
        acc_ref[0] = 0.0

    lab = lab_ref[0]
    labb = lab + jnp.zeros((T2, NC), jnp.float32)

    x = conf_cls_ref[0]
    p = 1.0 / (1.0 + jnp.exp(-x))
    col = jax.lax.broadcasted_iota(jnp.int32, (T2, NC), 1)
    col0 = col == 0
    colf = col.astype(jnp.float32)
    posf = jnp.clip(labb, 0.0, 1.0)
    eqf = (labb == colf).astype(jnp.float32)
    yf = jnp.where(col0, posf, eqf)
    negf = jnp.clip(1.0 - jnp.abs(labb), 0.0, 1.0)
    w = posf + jnp.where(col0, 0.5 * negf, 0.0)
    q = jnp.where(yf > 0.0, p, 1.0 - p)
    acc_ref[0] += jnp.sum(w * jnp.maximum(jnp.log(q), LOG_EPS))

    @pl.when(last)
    def _():
        v = scal_ref[...]
        num_pos = jnp.maximum(1.0, v[0:1, 1:2])
        out1_ref[...] = v[0:1, 0:1] / (num_pos * 4.0)
        out2_ref[...] = jnp.full((1, 1), -acc_ref[0]) / (2.0 * num_pos)


def kernel(confidence, predicted_locations, gts, counts, anchors):
    corners = jnp.concatenate(
        [anchors[:, :2] - anchors[:, 2:] * 0.5,
         anchors[:, :2] + anchors[:, 2:] * 0.5], axis=1)
    anchors8 = jnp.concatenate([corners.T, anchors.T], axis=0)
    ploc_l = jnp.swapaxes(predicted_locations, 1, 2)

    valid = (jnp.arange(NGT)[None, :] < counts[:, None])
    sent = jnp.array([-5.0, -5.0, -4.0, -4.0], jnp.float32)
    gbox = jnp.where(valid[:, :, None], gts[:, :, :4], sent[None, None, :])
    garea = ((gbox[:, :, 2] - gbox[:, :, 0])
             * (gbox[:, :, 3] - gbox[:, :, 1]) + 1e-9)
    glab = jnp.where(valid, gts[:, :, 4], 0.0)
    gt_t = jnp.concatenate(
        [jnp.swapaxes(gbox, 1, 2), garea[:, None, :], glab[:, None, :],
         jnp.zeros((B, 2, NGT), jnp.float32)], axis=1)
    gt_sc = jnp.pad(gt_t, ((0, 0), (0, 0), (0, NGT_PAD - NGT))
                    ).reshape(B, 8 * NGT_PAD)

    mesh = plsc.VectorSubcoreMesh(core_axis_name="c", subcore_axis_name="s",
                                  num_cores=NSC, num_subcores=NSUB)
    conf_l, mbox_l = pl.kernel(
        _sc_match,
        out_type=[
            jax.ShapeDtypeStruct((B, 1, A), jnp.float32),
            jax.ShapeDtypeStruct((B, 4, A), jnp.float32),
        ],
        mesh=mesh,
        scratch_types=[
            pltpu.VMEM((APW,), jnp.float32),
            pltpu.VMEM((APW,), jnp.float32),
            pltpu.VMEM((APW,), jnp.float32),
            pltpu.VMEM((APW,), jnp.float32),
            pltpu.VMEM((8 * NGT_PAD,), jnp.float32),
            pltpu.VMEM((APW,), jnp.float32),
            pltpu.VMEM((APW,), jnp.float32),
            pltpu.VMEM((APW,), jnp.float32),
            pltpu.VMEM((APW,), jnp.float32),
            pltpu.VMEM((APW,), jnp.float32),
        ],
    )(anchors8, gt_sc)

    scal = pl.pallas_call(
        _loc_body,
        grid=(B, A // T1),
        in_specs=[
            pl.BlockSpec((8, T1), lambda b, t: (0, t)),
            pl.BlockSpec((1, 4, T1), lambda b, t: (b, 0, t)),
            pl.BlockSpec((1, 1, T1), lambda b, t: (b, 0, t)),
            pl.BlockSpec((1, 4, T1), lambda b, t: (b, 0, t)),
        ],
        out_specs=pl.BlockSpec((1, 2), lambda b, t: (0, 0)),
        out_shape=jax.ShapeDtypeStruct((1, 2), jnp.float32),
        scratch_shapes=[pltpu.SMEM((2,), jnp.float32)],
        interpret=_INTERPRET,
    )(anchors8, ploc_l, conf_l, mbox_l)

    lab_s = jnp.swapaxes(conf_l, 1, 2)

    out1, out2 = pl.pallas_call(
        _loss_body,
        grid=(B, A // T2),
        in_specs=[
            pl.BlockSpec((1, T2, NC), lambda b, t: (b, t, 0)),
            pl.BlockSpec((1, T2, 1), lambda b, t: (b, t, 0)),
            pl.BlockSpec((1, 2), lambda b, t: (0, 0)),
        ],
        out_specs=[
            pl.BlockSpec((1, 1), lambda b, t: (0, 0)),
            pl.BlockSpec((1, 1), lambda b, t: (0, 0)),
        ],
        out_shape=[
            jax.ShapeDtypeStruct((1, 1), jnp.float32),
            jax.ShapeDtypeStruct((1, 1), jnp.float32),
        ],
        scratch_shapes=[pltpu.SMEM((1,), jnp.float32)],
        interpret=_INTERPRET,
    )(confidence, lab_s, scal)

    return out1[0, 0], out2[0, 0]


# speedup vs baseline: 1.0030x; 1.0030x over previous
import jax
import jax.numpy as jnp
from jax import lax
from jax.experimental import pallas as pl
from jax.experimental.pallas import tpu as pltpu
from jax.experimental.pallas import tpu_sc as plsc

POS_TH = 0.5
NEG_TH = 0.4
EPS = 1e-7
LOG_EPS = -16.11809565095832
BETA = 1.0 / 9.0


B, A = 8, 16384
T1 = 4096
T2 = 8192
NGT = 50
NC = 81

NSC, NSUB = 2, 16
NW = NSC * NSUB
WPB = NW // B
APW = A // WPB
LANE = 16
NGT_PAD = 64


def _sc_match(anch_hbm, gt_hbm, conf_hbm, mbox_hbm,
              ax1_v, ay1_v, ax2_v, ay2_v, gt_v,
              conf_v, m0_v, m1_v, m2_v, m3_v):
    cid = lax.axis_index("c")
    sid = lax.axis_index("s")
    wid = sid * NSC + cid
    b = wid // WPB
    base = (wid % WPB) * APW

    pltpu.sync_copy(anch_hbm.at[0, pl.ds(base, APW)], ax1_v)
    pltpu.sync_copy(anch_hbm.at[1, pl.ds(base, APW)], ay1_v)
    pltpu.sync_copy(anch_hbm.at[2, pl.ds(base, APW)], ax2_v)
    pltpu.sync_copy(anch_hbm.at[3, pl.ds(base, APW)], ay2_v)
    pltpu.sync_copy(gt_hbm.at[b], gt_v)

    gsc = []
    for r in range(6):
        chunks = [gt_v[pl.ds(r * NGT_PAD + c * LANE, LANE)]
                  for c in range(NGT_PAD // LANE)]
        gsc.append([chunks[g // LANE][g % LANE] for g in range(NGT)])

    def chunk(i, carry):
        for u in range(2):
            s = i * (2 * LANE) + u * LANE
            ax1 = ax1_v[pl.ds(s, LANE)]
            ay1 = ay1_v[pl.ds(s, LANE)]
            ax2 = ax2_v[pl.ds(s, LANE)]
            ay2 = ay2_v[pl.ds(s, LANE)]
            area_a = (ax2 - ax1) * (ay2 - ay1)

            z = jnp.zeros((LANE,), jnp.float32)
            best = jnp.full((LANE,), -1.0, jnp.float32)
            labv = z
            mx1 = z
            my1 = z
            mx2 = z
            my2 = z
            for g in range(NGT):
                gx1 = gsc[0][g]
                gy1 = gsc[1][g]
                gx2 = gsc[2][g]
                gy2 = gsc[3][g]
                garea = gsc[4][g]
                glab = gsc[5][g]
                wx = jnp.maximum(jnp.minimum(ax2, gx2) - jnp.maximum(ax1, gx1),
                                 0.0)
                wy = jnp.maximum(jnp.minimum(ay2, gy2) - jnp.maximum(ay1, gy1),
                                 0.0)
                inter = wx * wy
                iou = inter / (area_a + (garea - inter))
                upd = iou > best
                best = jnp.where(upd, iou, best)
                labv = jnp.where(upd, glab, labv)
                mx1 = jnp.where(upd, gx1, mx1)
                my1 = jnp.where(upd, gy1, my1)
                mx2 = jnp.where(upd, gx2, mx2)
                my2 = jnp.where(upd, gy2, my2)

            conf = jnp.where(best < POS_TH, -1.0, labv)
            conf = jnp.where(best < NEG_TH, 0.0, conf)
            conf_v[pl.ds(s, LANE)] = conf
            m0_v[pl.ds(s, LANE)] = mx1
            m1_v[pl.ds(s, LANE)] = my1
            m2_v[pl.ds(s, LANE)] = mx2
            m3_v[pl.ds(s, LANE)] = my2
        return carry

    lax.fori_loop(0, APW // (2 * LANE), chunk, 0)

    pltpu.sync_copy(conf_v, conf_hbm.at[b, 0, pl.ds(base, APW)])
    pltpu.sync_copy(m0_v, mbox_hbm.at[b, 0, pl.ds(base, APW)])
    pltpu.sync_copy(m1_v, mbox_hbm.at[b, 1, pl.ds(base, APW)])
    pltpu.sync_copy(m2_v, mbox_hbm.at[b, 2, pl.ds(base, APW)])
    pltpu.sync_copy(m3_v, mbox_hbm.at[b, 3, pl.ds(base, APW)])


def _loc_body(a_ref, ploc_ref, conf_ref, mbox_ref, scal_ref, acc_ref):
    b = pl.program_id(0)
    t = pl.program_id(1)
    first = jnp.logical_and(b == 0, t == 0)
    last = jnp.logical_and(b == pl.num_programs(0) - 1,
                           t == pl.num_programs(1) - 1)

    @pl.when(first)
    def _():
        acc_ref[0] = 0.0
        acc_ref[1] = 0.0

    conf = conf_ref[0, 0:1, :]
    pos = (conf > 0.0).astype(jnp.float32)
    mx1 = mbox_ref[0, 0:1, :]
    my1 = mbox_ref[0, 1:2, :]
    mx2 = mbox_ref[0, 2:3, :]
    my2 = mbox_ref[0, 3:4, :]
    acx = a_ref[4:5, :]
    acy = a_ref[5:6, :]
    aw = a_ref[6:7, :]
    ah = a_ref[7:8, :]
    l0 = ((mx1 + mx2) * 0.5 - acx) / (aw * 0.1)
    l1 = ((my1 + my2) * 0.5 - acy) / (ah * 0.1)
    l2 = jnp.log(jnp.maximum(mx2 - mx1, 1e-6) / aw) * 5.0
    l3 = jnp.log(jnp.maximum(my2 - my1, 1e-6) / ah) * 5.0

    sl1 = jnp.zeros((1, T1), jnp.float32)
    for i, l in enumerate((l0, l1, l2, l3)):
        n = jnp.abs(ploc_ref[0, i:i + 1, :] - l)
        sl1 = sl1 + jnp.where(n < BETA, n * n * (0.5 / BETA), n - 0.5 * BETA)
    acc_ref[0] += jnp.sum(sl1 * pos)
    acc_ref[1] += jnp.sum(pos)

    @pl.when(last)
    def _():
        scal_ref[0:1, 0:1] = jnp.full((1, 1), acc_ref[0])
        scal_ref[0:1, 1:2] = jnp.full((1, 1), acc_ref[1])


def _loss_body(conf_cls_ref, lab_ref, scal_ref, out1_ref, out2_ref, acc_ref):
    b = pl.program_id(0)
    t = pl.program_id(1)
    first = jnp.logical_and(b == 0, t == 0)
    last = jnp.logical_and(b == pl.num_programs(0) - 1,
                           t == pl.num_programs(1) - 1)

    @pl.when(first)
    def _():
        acc_ref[0] = 0.0

    lab = lab_ref[0]
    labb = lab + jnp.zeros((T2, NC), jnp.float32)

    x = conf_cls_ref[0]
    p = 1.0 / (1.0 + jnp.exp(-x))
    col = jax.lax.broadcasted_iota(jnp.int32, (T2, NC), 1)
    col0 = col == 0
    colf = col.astype(jnp.float32)
    posf = jnp.clip(labb, 0.0, 1.0)
    eqf = (labb == colf).astype(jnp.float32)
    yf = jnp.where(col0, posf, eqf)
    negf = jnp.clip(1.0 - jnp.abs(labb), 0.0, 1.0)
    w = posf + jnp.where(col0, 0.5 * negf, 0.0)
    q = jnp.where(yf > 0.0, p, 1.0 - p)
    acc_ref[0] += jnp.sum(w * jnp.maximum(jnp.log(q), LOG_EPS))

    @pl.when(last)
    def _():
        v = scal_ref[...]
        num_pos = jnp.maximum(1.0, v[0:1, 1:2])
        out1_ref[...] = v[0:1, 0:1] / (num_pos * 4.0)
        out2_ref[...] = jnp.full((1, 1), -acc_ref[0]) / (2.0 * num_pos)


def kernel(confidence, predicted_locations, gts, counts, anchors):
    corners = jnp.concatenate(
        [anchors[:, :2] - anchors[:, 2:] * 0.5,
         anchors[:, :2] + anchors[:, 2:] * 0.5], axis=1)
    anchors8 = jnp.concatenate([corners.T, anchors.T], axis=0)
    ploc_l = jnp.swapaxes(predicted_locations, 1, 2)

    valid = (jnp.arange(NGT)[None, :] < counts[:, None])
    sent = jnp.array([-5.0, -5.0, -4.0, -4.0], jnp.float32)
    gbox = jnp.where(valid[:, :, None], gts[:, :, :4], sent[None, None, :])
    garea = ((gbox[:, :, 2] - gbox[:, :, 0])
             * (gbox[:, :, 3] - gbox[:, :, 1]) + 1e-9)
    glab = jnp.where(valid, gts[:, :, 4], 0.0)
    gt_t = jnp.concatenate(
        [jnp.swapaxes(gbox, 1, 2), garea[:, None, :], glab[:, None, :],
         jnp.zeros((B, 2, NGT), jnp.float32)], axis=1)
    gt_sc = jnp.pad(gt_t, ((0, 0), (0, 0), (0, NGT_PAD - NGT))
                    ).reshape(B, 8 * NGT_PAD)

    mesh = plsc.VectorSubcoreMesh(core_axis_name="c", subcore_axis_name="s",
                                  num_cores=NSC, num_subcores=NSUB)
    conf_l, mbox_l = pl.kernel(
        _sc_match,
        out_type=[
            jax.ShapeDtypeStruct((B, 1, A), jnp.float32),
            jax.ShapeDtypeStruct((B, 4, A), jnp.float32),
        ],
        mesh=mesh,
        scratch_types=[
            pltpu.VMEM((APW,), jnp.float32),
            pltpu.VMEM((APW,), jnp.float32),
            pltpu.VMEM((APW,), jnp.float32),
            pltpu.VMEM((APW,), jnp.float32),
            pltpu.VMEM((8 * NGT_PAD,), jnp.float32),
            pltpu.VMEM((APW,), jnp.float32),
            pltpu.VMEM((APW,), jnp.float32),
            pltpu.VMEM((APW,), jnp.float32),
            pltpu.VMEM((APW,), jnp.float32),
            pltpu.VMEM((APW,), jnp.float32),
        ],
    )(anchors8, gt_sc)

    scal = pl.pallas_call(
        _loc_body,
        grid=(B, A // T1),
        in_specs=[
            pl.BlockSpec((8, T1), lambda b, t: (0, t)),
            pl.BlockSpec((1, 4, T1), lambda b, t: (b, 0, t)),
            pl.BlockSpec((1, 1, T1), lambda b, t: (b, 0, t)),
            pl.BlockSpec((1, 4, T1), lambda b, t: (b, 0, t)),
        ],
        out_specs=pl.BlockSpec((1, 2), lambda b, t: (0, 0)),
        out_shape=jax.ShapeDtypeStruct((1, 2), jnp.float32),
        scratch_shapes=[pltpu.SMEM((2,), jnp.float32)],
    )(anchors8, ploc_l, conf_l, mbox_l)

    lab_s = jnp.swapaxes(conf_l, 1, 2)

    out1, out2 = pl.pallas_call(
        _loss_body,
        grid=(B, A // T2),
        in_specs=[
            pl.BlockSpec((1, T2, NC), lambda b, t: (b, t, 0)),
            pl.BlockSpec((1, T2, 1), lambda b, t: (b, t, 0)),
            pl.BlockSpec((1, 2), lambda b, t: (0, 0)),
        ],
        out_specs=[
            pl.BlockSpec((1, 1), lambda b, t: (0, 0)),
            pl.BlockSpec((1, 1), lambda b, t: (0, 0)),
        ],
        out_shape=[
            jax.ShapeDtypeStruct((1, 1), jnp.float32),
            jax.ShapeDtypeStruct((1, 1), jnp.float32),
        ],
        scratch_shapes=[pltpu.SMEM((1,), jnp.float32)],
    )(confidence, lab_s, scal)

    return out1[0, 0], out2[0, 0]
